# baseline (speedup 1.0000x reference)
import functools

import jax
import jax.numpy as jnp
from jax import lax
from jax.experimental import pallas as pl
from jax.experimental.pallas import tpu as pltpu
from jax.experimental.pallas import tpu_sc as plsc

N = 10000
K = 10
D = 128
D_EDGE = 20

BS = 400
NB = N // BS
NC, NS = 2, 16
NW = NC * NS

NPAD = 10240
JPW = NPAD // NW
JPC = 8
CH1 = JPC * K
NCH1 = JPW // JPC

B_PAD = NPAD * K
PER_W = B_PAD // NW
CH2 = 80
NCH2 = PER_W // CH2
NBUF = 4


def _cos_poly(x):
    x2 = x * x
    return 1.0 + x2 * (-0.5 + x2 * (1.0 / 24 + x2 * (-1.0 / 720
                                                     + x2 * (1.0 / 40320))))


def _sin_poly(x):
    x2 = x * x
    return x * (1.0 + x2 * (-1.0 / 6 + x2 * (1.0 / 120 + x2 * (-1.0 / 5040
                                                               + x2 * (1.0 / 362880)))))


def _sc_gather_sum(table, idx2d):
    mesh = plsc.VectorSubcoreMesh(core_axis_name="c", subcore_axis_name="s")

    @functools.partial(
        pl.kernel,
        mesh=mesh,
        out_type=jax.ShapeDtypeStruct((NPAD, D), jnp.float32),
        scratch_types=[
            pltpu.VMEM((NCH1, CH1), jnp.int32),
            *[pltpu.VMEM((CH1, D), jnp.float32) for _ in range(NBUF)],
            *[pltpu.VMEM((JPC, D), jnp.float32) for _ in range(NBUF)],
            *[pltpu.SemaphoreType.DMA for _ in range(2 * NBUF)],
        ],
    )
    def gk(x_hbm, idx_hbm, g_hbm, idx_v, *bufs):
        rows = bufs[:NBUF]
        gbuf = bufs[NBUF:2 * NBUF]
        gsems = bufs[2 * NBUF:3 * NBUF]
        ssems = bufs[3 * NBUF:]
        wid = lax.axis_index("s") * NC + lax.axis_index("c")
        pltpu.sync_copy(
            idx_hbm.at[pl.ds(pl.multiple_of(wid * NCH1, 8), NCH1)], idx_v)
        for b in range(NBUF):
            pltpu.async_copy(x_hbm.at[idx_v.at[b]], rows[b], gsems[b])

        def outer(g, carry):
            for b in range(NBUF):
                ci = g * NBUF + b
                pltpu.make_async_copy(
                    x_hbm.at[pl.ds(0, CH1)], rows[b], gsems[b]).wait()

                @pl.when(g > 0)
                def _():
                    pltpu.make_async_copy(
                        gbuf[b], g_hbm.at[pl.ds(0, JPC)], ssems[b]).wait()

                for jl in range(JPC):
                    for cc in range(D // 16):
                        sl = pl.ds(cc * 16, 16)
                        acc = rows[b][jl * K, sl]
                        for kk in range(1, K):
                            acc = acc + rows[b][jl * K + kk, sl]
                        gbuf[b][jl, sl] = acc
                pltpu.async_copy(
                    gbuf[b],
                    g_hbm.at[pl.ds(pl.multiple_of(wid * JPW + ci * JPC, 8),
                                   JPC)],
                    ssems[b])
                nci = ci + NBUF

                @pl.when(nci < NCH1)
                def _():
                    pltpu.async_copy(
                        x_hbm.at[idx_v.at[nci]], rows[b], gsems[b])
            return carry

        lax.fori_loop(0, NCH1 // NBUF, outer, 0)
        for b in range(NBUF):
            pltpu.make_async_copy(
                gbuf[b], g_hbm.at[pl.ds(0, JPC)], ssems[b]).wait()

    return gk(table, idx2d)


def _sc_gather(table, idx1d):
    Wd = table.shape[1]
    mesh = plsc.VectorSubcoreMesh(core_axis_name="c", subcore_axis_name="s")

    @functools.partial(
        pl.kernel,
        mesh=mesh,
        out_type=jax.ShapeDtypeStruct((B_PAD, Wd), jnp.float32),
        scratch_types=[
            pltpu.VMEM((PER_W,), jnp.int32),
            *[pltpu.VMEM((CH2, Wd), jnp.float32) for _ in range(NBUF)],
            *[pltpu.SemaphoreType.DMA for _ in range(2 * NBUF)],
        ],
    )
    def gk(t_hbm, idx_hbm, out_hbm, idx_v, *bufs):
        rows = bufs[:NBUF]
        gsems = bufs[NBUF:2 * NBUF]
        ssems = bufs[2 * NBUF:]
        wid = lax.axis_index("s") * NC + lax.axis_index("c")
        base = wid * PER_W
        pltpu.sync_copy(
            idx_hbm.at[pl.ds(pl.multiple_of(wid * PER_W, 8), PER_W)], idx_v)
        for b in range(2):
            pltpu.async_copy(
                t_hbm.at[idx_v.at[pl.ds(b * CH2, CH2)]], rows[b], gsems[b])

        def outer(g, carry):
            for b in range(NBUF):
                ci = g * NBUF + b
                b2 = (b + 2) % NBUF
                nci = ci + 2
                pltpu.make_async_copy(
                    t_hbm.at[pl.ds(0, CH2)], rows[b], gsems[b]).wait()
                pltpu.async_copy(
                    rows[b],
                    out_hbm.at[pl.ds(pl.multiple_of(base + ci * CH2, 8), CH2)],
                    ssems[b])

                @pl.when(jnp.logical_and(nci >= NBUF, nci < NCH2))
                def _():
                    pltpu.make_async_copy(
                        rows[b2], out_hbm.at[pl.ds(0, CH2)], ssems[b2]).wait()
                    pltpu.async_copy(
                        t_hbm.at[idx_v.at[pl.ds(pl.multiple_of(nci * CH2, 8),
                                                CH2)]],
                        rows[b2], gsems[b2])

                @pl.when(nci < NBUF)
                def _():
                    pltpu.async_copy(
                        t_hbm.at[idx_v.at[pl.ds(pl.multiple_of(nci * CH2, 8),
                                                CH2)]],
                        rows[b2], gsems[b2])
            return carry

        lax.fori_loop(0, NCH2 // NBUF, outer, 0)
        for b in range(NBUF):
            pltpu.make_async_copy(
                rows[b], out_hbm.at[pl.ds(0, CH2)], ssems[b]).wait()

    return gk(table, idx1d)


def _tables_kernel(G_ref, et_ref, ef_ref, t_ref, w_ref, b_ref, A1_ref, C1_ref,
                   b1_ref, T_ref, c_ref, s_ref, E_ref):
    w = w_ref[...]
    Cacc = jnp.zeros((BS, D), jnp.float32)
    Sacc = jnp.zeros((BS, D), jnp.float32)
    for kk in range(K):
        ang = et_ref[:, kk:kk + 1] * w
        Cacc = Cacc + _cos_poly(ang)
        Sacc = Sacc + _sin_poly(ang)
    Eacc = jnp.zeros((BS, D_EDGE), jnp.float32)
    for kk in range(K):
        Eacc = Eacc + ef_ref[:, kk * D_EDGE:(kk + 1) * D_EDGE]
    P = (jnp.dot(G_ref[...], A1_ref[...], preferred_element_type=jnp.float32)
         + jnp.dot(Eacc, C1_ref[...], preferred_element_type=jnp.float32)
         + float(K) * b1_ref[...])
    T_ref[:, :D] = Cacc.astype(jnp.bfloat16)
    T_ref[:, D:2 * D] = Sacc.astype(jnp.bfloat16)
    T_ref[:, 2 * D:3 * D] = P.astype(jnp.bfloat16)
    T_ref[:, 3 * D:] = jnp.zeros((BS, D), jnp.bfloat16)
    phase = t_ref[...] * w + b_ref[...]
    c_ref[...] = _cos_poly(phase)
    s_ref[...] = _sin_poly(phase)
    E_ref[...] = Eacc


def _finish_kernel(Tg_ref, T_ref, c_ref, s_ref, G_ref, E_ref, x_ref,
                   B1_ref, WaA2_ref, WbA2_ref, B2_ref, C2_ref, W2d_ref,
                   W2e_ref, const2_ref, fc_ref, out_ref):
    c = c_ref[...]
    s = s_ref[...]
    c4 = jnp.broadcast_to(c[:, None, :], (BS, K, D)).reshape(BS * K, D)
    s4 = jnp.broadcast_to(s[:, None, :], (BS, K, D)).reshape(BS * K, D)
    Cg = Tg_ref[:, :D].astype(jnp.float32)
    Sg = Tg_ref[:, D:2 * D].astype(jnp.float32)
    Pg = Tg_ref[:, 2 * D:3 * D].astype(jnp.float32)
    vm = c4 * Cg + s4 * Sg
    u = jnp.maximum(
        Pg + jnp.dot(vm, B1_ref[...], preferred_element_type=jnp.float32), 0.0)
    U = jnp.sum(u.reshape(BS, K, D), axis=1)
    tt = (c * T_ref[:, :D].astype(jnp.float32)
          + s * T_ref[:, D:2 * D].astype(jnp.float32))
    pre = (jnp.dot(U, WaA2_ref[...], preferred_element_type=jnp.float32)
           + jnp.dot(G_ref[...], WbA2_ref[...], preferred_element_type=jnp.float32)
           + jnp.dot(tt, B2_ref[...], preferred_element_type=jnp.float32)
           + jnp.dot(E_ref[...], C2_ref[...], preferred_element_type=jnp.float32)
           + const2_ref[...])
    out_ref[...] = (
        jnp.dot(jnp.maximum(pre, 0.0), W2d_ref[...],
                preferred_element_type=jnp.float32)
        + jnp.dot(x_ref[...], W2e_ref[...], preferred_element_type=jnp.float32)
        + fc_ref[...])


def kernel(x, t, neighbor_idx, edge_times, edge_feats, time_w, time_b, W1, b1, W2, b2):
    nbr = neighbor_idx.astype(jnp.int32)
    pad = jnp.zeros((NPAD * K - N * K,), jnp.int32)
    idx_flat = jnp.concatenate([nbr.reshape(-1), pad])
    idx_j = idx_flat.reshape(NW * NCH1, CH1)

    A1, B1w, C1 = W1[0][:D], W1[0][D:2 * D], W1[0][2 * D:]
    A2, B2w, C2 = W1[1][:D], W1[1][D:2 * D], W1[1][2 * D:]
    W2a, W2b, W2c = W2[0][:D], W2[0][D:2 * D], W2[0][2 * D:]
    W2d, W2e, W2f = W2[1][:D], W2[1][D:2 * D], W2[1][2 * D:]
    z = jnp.cos(time_b)
    cr = z @ W2c + b2[0]
    WaA2 = W2a @ A2
    WbA2 = W2b @ A2
    const2 = (float(K) * (cr @ A2 + b1[1])).reshape(1, D)
    fc = (z @ W2f + b2[1]).reshape(1, D)

    G = _sc_gather_sum(x, idx_j)

    ef2 = edge_feats.reshape(N, K * D_EDGE)
    T, c, s, E = pl.pallas_call(
        _tables_kernel,
        grid=(NB,),
        in_specs=[
            pl.BlockSpec((BS, D), lambda i: (i, 0)),
            pl.BlockSpec((BS, K), lambda i: (i, 0)),
            pl.BlockSpec((BS, K * D_EDGE), lambda i: (i, 0)),
            pl.BlockSpec((BS, 1), lambda i: (i, 0)),
            pl.BlockSpec((1, D), lambda i: (0, 0)),
            pl.BlockSpec((1, D), lambda i: (0, 0)),
            pl.BlockSpec((D, D), lambda i: (0, 0)),
            pl.BlockSpec((D_EDGE, D), lambda i: (0, 0)),
            pl.BlockSpec((1, D), lambda i: (0, 0)),
        ],
        out_specs=[
            pl.BlockSpec((BS, 4 * D), lambda i: (i, 0)),
            pl.BlockSpec((BS, D), lambda i: (i, 0)),
            pl.BlockSpec((BS, D), lambda i: (i, 0)),
            pl.BlockSpec((BS, D_EDGE), lambda i: (i, 0)),
        ],
        out_shape=[
            jax.ShapeDtypeStruct((N, 4 * D), jnp.bfloat16),
            jax.ShapeDtypeStruct((N, D), jnp.float32),
            jax.ShapeDtypeStruct((N, D), jnp.float32),
            jax.ShapeDtypeStruct((N, D_EDGE), jnp.float32),
        ],
        compiler_params=pltpu.CompilerParams(
            dimension_semantics=("arbitrary",)),
    )(G, edge_times, ef2, t.reshape(N, 1), time_w.reshape(1, D),
      time_b.reshape(1, D), A1, C1, b1[0].reshape(1, D))

    Tp = jax.lax.bitcast_convert_type(
        T.reshape(N, 2 * D, 2), jnp.float32)
    Tg = _sc_gather(Tp, idx_flat)
    Tg_bf = jax.lax.bitcast_convert_type(
        Tg, jnp.bfloat16).reshape(B_PAD, 4 * D)

    out = pl.pallas_call(
        _finish_kernel,
        grid=(NB,),
        in_specs=[
            pl.BlockSpec((BS * K, 4 * D), lambda i: (i, 0)),
            pl.BlockSpec((BS, 4 * D), lambda i: (i, 0)),
            pl.BlockSpec((BS, D), lambda i: (i, 0)),
            pl.BlockSpec((BS, D), lambda i: (i, 0)),
            pl.BlockSpec((BS, D), lambda i: (i, 0)),
            pl.BlockSpec((BS, D_EDGE), lambda i: (i, 0)),
            pl.BlockSpec((BS, D), lambda i: (i, 0)),
            pl.BlockSpec((D, D), lambda i: (0, 0)),
            pl.BlockSpec((D, D), lambda i: (0, 0)),
            pl.BlockSpec((D, D), lambda i: (0, 0)),
            pl.BlockSpec((D, D), lambda i: (0, 0)),
            pl.BlockSpec((D_EDGE, D), lambda i: (0, 0)),
            pl.BlockSpec((D, D), lambda i: (0, 0)),
            pl.BlockSpec((D, D), lambda i: (0, 0)),
            pl.BlockSpec((1, D), lambda i: (0, 0)),
            pl.BlockSpec((1, D), lambda i: (0, 0)),
        ],
        out_specs=pl.BlockSpec((BS, D), lambda i: (i, 0)),
        out_shape=jax.ShapeDtypeStruct((N, D), jnp.float32),
        compiler_params=pltpu.CompilerParams(
            dimension_semantics=("arbitrary",)),
    )(Tg_bf, T, c, s, G, E, x, B1w, WaA2, WbA2, B2w, C2, W2d, W2e, const2, fc)

    return out


# speedup vs baseline: 2.8711x; 2.8711x over previous
import functools

import jax
import jax.numpy as jnp
from jax import lax
from jax.experimental import pallas as pl
from jax.experimental.pallas import tpu as pltpu
from jax.experimental.pallas import tpu_sc as plsc

N = 10000
K = 10
D = 128
D_EDGE = 20

BS = 400
NB = N // BS
NC, NS = 2, 16
NW = NC * NS

NPAD = 10240
JPW = NPAD // NW
JPC = 8
CH1 = JPC * K
NCH1 = JPW // JPC

B_PAD = NPAD * K
PER_W = B_PAD // NW
CH2 = 80
NCH2 = PER_W // CH2
NBUF = 4


def _cos_poly(x):
    x2 = x * x
    return 1.0 + x2 * (-0.5 + x2 * (1.0 / 24 + x2 * (-1.0 / 720
                                                     + x2 * (1.0 / 40320))))


def _sin_poly(x):
    x2 = x * x
    return x * (1.0 + x2 * (-1.0 / 6 + x2 * (1.0 / 120 + x2 * (-1.0 / 5040
                                                               + x2 * (1.0 / 362880)))))


def _sc_gather_sum(table, idx2d):
    mesh = plsc.VectorSubcoreMesh(core_axis_name="c", subcore_axis_name="s")

    @functools.partial(
        pl.kernel,
        mesh=mesh,
        out_type=jax.ShapeDtypeStruct((NPAD, D), jnp.float32),
        scratch_types=[
            pltpu.VMEM((NCH1, CH1), jnp.int32),
            *[pltpu.VMEM((CH1, D), jnp.float32) for _ in range(NBUF)],
            *[pltpu.VMEM((JPC, D), jnp.float32) for _ in range(NBUF)],
            *[pltpu.SemaphoreType.DMA for _ in range(2 * NBUF)],
        ],
    )
    def gk(x_hbm, idx_hbm, g_hbm, idx_v, *bufs):
        rows = bufs[:NBUF]
        gbuf = bufs[NBUF:2 * NBUF]
        gsems = bufs[2 * NBUF:3 * NBUF]
        ssems = bufs[3 * NBUF:]
        wid = lax.axis_index("s") * NC + lax.axis_index("c")
        pltpu.sync_copy(
            idx_hbm.at[pl.ds(pl.multiple_of(wid * NCH1, 8), NCH1)], idx_v)
        for b in range(NBUF):
            pltpu.async_copy(x_hbm.at[idx_v.at[b]], rows[b], gsems[b])

        def outer(g, carry):
            for b in range(NBUF):
                ci = g * NBUF + b
                pltpu.make_async_copy(
                    x_hbm.at[pl.ds(0, CH1)], rows[b], gsems[b]).wait()

                @pl.when(g > 0)
                def _():
                    pltpu.make_async_copy(
                        gbuf[b], g_hbm.at[pl.ds(0, JPC)], ssems[b]).wait()

                for jl in range(JPC):
                    for cc in range(D // 16):
                        sl = pl.ds(cc * 16, 16)
                        acc = rows[b][jl * K, sl]
                        for kk in range(1, K):
                            acc = acc + rows[b][jl * K + kk, sl]
                        gbuf[b][jl, sl] = acc
                pltpu.async_copy(
                    gbuf[b],
                    g_hbm.at[pl.ds(pl.multiple_of(wid * JPW + ci * JPC, 8),
                                   JPC)],
                    ssems[b])
                nci = ci + NBUF

                @pl.when(nci < NCH1)
                def _():
                    pltpu.async_copy(
                        x_hbm.at[idx_v.at[nci]], rows[b], gsems[b])
            return carry

        lax.fori_loop(0, NCH1 // NBUF, outer, 0)
        for b in range(NBUF):
            pltpu.make_async_copy(
                gbuf[b], g_hbm.at[pl.ds(0, JPC)], ssems[b]).wait()

    return gk(table, idx2d)


def _sc_gather(table, idx1d):
    Wd = table.shape[1]
    mesh = plsc.VectorSubcoreMesh(core_axis_name="c", subcore_axis_name="s")

    @functools.partial(
        pl.kernel,
        mesh=mesh,
        out_type=jax.ShapeDtypeStruct((B_PAD, Wd), jnp.float32),
        scratch_types=[
            pltpu.VMEM((PER_W,), jnp.int32),
            *[pltpu.VMEM((CH2, Wd), jnp.float32) for _ in range(NBUF)],
            *[pltpu.SemaphoreType.DMA for _ in range(2 * NBUF)],
        ],
    )
    def gk(t_hbm, idx_hbm, out_hbm, idx_v, *bufs):
        rows = bufs[:NBUF]
        gsems = bufs[NBUF:2 * NBUF]
        ssems = bufs[2 * NBUF:]
        wid = lax.axis_index("s") * NC + lax.axis_index("c")
        base = wid * PER_W
        pltpu.sync_copy(
            idx_hbm.at[pl.ds(pl.multiple_of(wid * PER_W, 8), PER_W)], idx_v)
        for b in range(2):
            pltpu.async_copy(
                t_hbm.at[idx_v.at[pl.ds(b * CH2, CH2)]], rows[b], gsems[b])

        def outer(g, carry):
            for b in range(NBUF):
                ci = g * NBUF + b
                b2 = (b + 2) % NBUF
                nci = ci + 2
                pltpu.make_async_copy(
                    t_hbm.at[pl.ds(0, CH2)], rows[b], gsems[b]).wait()
                pltpu.async_copy(
                    rows[b],
                    out_hbm.at[pl.ds(pl.multiple_of(base + ci * CH2, 8), CH2)],
                    ssems[b])

                @pl.when(jnp.logical_and(nci >= NBUF, nci < NCH2))
                def _():
                    pltpu.make_async_copy(
                        rows[b2], out_hbm.at[pl.ds(0, CH2)], ssems[b2]).wait()
                    pltpu.async_copy(
                        t_hbm.at[idx_v.at[pl.ds(pl.multiple_of(nci * CH2, 8),
                                                CH2)]],
                        rows[b2], gsems[b2])

                @pl.when(nci < NBUF)
                def _():
                    pltpu.async_copy(
                        t_hbm.at[idx_v.at[pl.ds(pl.multiple_of(nci * CH2, 8),
                                                CH2)]],
                        rows[b2], gsems[b2])
            return carry

        lax.fori_loop(0, NCH2 // NBUF, outer, 0)
        for b in range(NBUF):
            pltpu.make_async_copy(
                rows[b], out_hbm.at[pl.ds(0, CH2)], ssems[b]).wait()

    return gk(table, idx1d)


def _tables_kernel(G_ref, et_ref, ef_ref, t_ref, w_ref, b_ref, A1_ref, C1_ref,
                   b1_ref, T_ref, c_ref, s_ref, E_ref):
    w = w_ref[...]
    Cacc = jnp.zeros((BS, D), jnp.float32)
    Sacc = jnp.zeros((BS, D), jnp.float32)
    for kk in range(K):
        ang = et_ref[:, kk:kk + 1] * w
        Cacc = Cacc + _cos_poly(ang)
        Sacc = Sacc + _sin_poly(ang)
    Eacc = jnp.zeros((BS, D_EDGE), jnp.float32)
    for kk in range(K):
        Eacc = Eacc + ef_ref[:, kk * D_EDGE:(kk + 1) * D_EDGE]
    P = (jnp.dot(G_ref[...], A1_ref[...], preferred_element_type=jnp.float32)
         + jnp.dot(Eacc, C1_ref[...], preferred_element_type=jnp.float32)
         + float(K) * b1_ref[...])
    cu = ((jax.lax.bitcast_convert_type(Cacc, jnp.uint32) + jnp.uint32(0x8000))
          & jnp.uint32(0xFFFF0000))
    su = ((jax.lax.bitcast_convert_type(Sacc, jnp.uint32) + jnp.uint32(0x8000))
          >> 16)
    T_ref[:, :D] = jax.lax.bitcast_convert_type(cu | su, jnp.float32)
    T_ref[:, D:] = P
    phase = t_ref[...] * w + b_ref[...]
    c_ref[...] = _cos_poly(phase)
    s_ref[...] = _sin_poly(phase)
    E_ref[...] = Eacc


def _finish_kernel(Tg_ref, T_ref, c_ref, s_ref, G_ref, E_ref, x_ref,
                   B1_ref, WaA2_ref, WbA2_ref, B2_ref, C2_ref, W2d_ref,
                   W2e_ref, const2_ref, fc_ref, out_ref):
    c = c_ref[...]
    s = s_ref[...]
    c4 = jnp.broadcast_to(c[:, None, :], (BS, K, D)).reshape(BS * K, D)
    s4 = jnp.broadcast_to(s[:, None, :], (BS, K, D)).reshape(BS * K, D)
    wg = jax.lax.bitcast_convert_type(Tg_ref[:, :D], jnp.uint32)
    Cg = jax.lax.bitcast_convert_type(wg & jnp.uint32(0xFFFF0000), jnp.float32)
    Sg = jax.lax.bitcast_convert_type(wg << 16, jnp.float32)
    Pg = Tg_ref[:, D:]
    vm = c4 * Cg + s4 * Sg
    u = jnp.maximum(
        Pg + jnp.dot(vm, B1_ref[...], preferred_element_type=jnp.float32), 0.0)
    U = jnp.sum(u.reshape(BS, K, D), axis=1)
    wt = jax.lax.bitcast_convert_type(T_ref[:, :D], jnp.uint32)
    tt = (c * jax.lax.bitcast_convert_type(wt & jnp.uint32(0xFFFF0000),
                                           jnp.float32)
          + s * jax.lax.bitcast_convert_type(wt << 16, jnp.float32))
    pre = (jnp.dot(U, WaA2_ref[...], preferred_element_type=jnp.float32)
           + jnp.dot(G_ref[...], WbA2_ref[...], preferred_element_type=jnp.float32)
           + jnp.dot(tt, B2_ref[...], preferred_element_type=jnp.float32)
           + jnp.dot(E_ref[...], C2_ref[...], preferred_element_type=jnp.float32)
           + const2_ref[...])
    out_ref[...] = (
        jnp.dot(jnp.maximum(pre, 0.0), W2d_ref[...],
                preferred_element_type=jnp.float32)
        + jnp.dot(x_ref[...], W2e_ref[...], preferred_element_type=jnp.float32)
        + fc_ref[...])


def kernel(x, t, neighbor_idx, edge_times, edge_feats, time_w, time_b, W1, b1, W2, b2):
    nbr = neighbor_idx.astype(jnp.int32)
    pad = jnp.zeros((NPAD * K - N * K,), jnp.int32)
    idx_flat = jnp.concatenate([nbr.reshape(-1), pad])
    idx_j = idx_flat.reshape(NW * NCH1, CH1)

    A1, B1w, C1 = W1[0][:D], W1[0][D:2 * D], W1[0][2 * D:]
    A2, B2w, C2 = W1[1][:D], W1[1][D:2 * D], W1[1][2 * D:]
    W2a, W2b, W2c = W2[0][:D], W2[0][D:2 * D], W2[0][2 * D:]
    W2d, W2e, W2f = W2[1][:D], W2[1][D:2 * D], W2[1][2 * D:]
    z = jnp.cos(time_b)
    cr = z @ W2c + b2[0]
    WaA2 = W2a @ A2
    WbA2 = W2b @ A2
    const2 = (float(K) * (cr @ A2 + b1[1])).reshape(1, D)
    fc = (z @ W2f + b2[1]).reshape(1, D)

    G = _sc_gather_sum(x, idx_j)

    ef2 = edge_feats.reshape(N, K * D_EDGE)
    T, c, s, E = pl.pallas_call(
        _tables_kernel,
        grid=(NB,),
        in_specs=[
            pl.BlockSpec((BS, D), lambda i: (i, 0)),
            pl.BlockSpec((BS, K), lambda i: (i, 0)),
            pl.BlockSpec((BS, K * D_EDGE), lambda i: (i, 0)),
            pl.BlockSpec((BS, 1), lambda i: (i, 0)),
            pl.BlockSpec((1, D), lambda i: (0, 0)),
            pl.BlockSpec((1, D), lambda i: (0, 0)),
            pl.BlockSpec((D, D), lambda i: (0, 0)),
            pl.BlockSpec((D_EDGE, D), lambda i: (0, 0)),
            pl.BlockSpec((1, D), lambda i: (0, 0)),
        ],
        out_specs=[
            pl.BlockSpec((BS, 2 * D), lambda i: (i, 0)),
            pl.BlockSpec((BS, D), lambda i: (i, 0)),
            pl.BlockSpec((BS, D), lambda i: (i, 0)),
            pl.BlockSpec((BS, D_EDGE), lambda i: (i, 0)),
        ],
        out_shape=[
            jax.ShapeDtypeStruct((N, 2 * D), jnp.float32),
            jax.ShapeDtypeStruct((N, D), jnp.float32),
            jax.ShapeDtypeStruct((N, D), jnp.float32),
            jax.ShapeDtypeStruct((N, D_EDGE), jnp.float32),
        ],
        compiler_params=pltpu.CompilerParams(
            dimension_semantics=("arbitrary",)),
    )(G, edge_times, ef2, t.reshape(N, 1), time_w.reshape(1, D),
      time_b.reshape(1, D), A1, C1, b1[0].reshape(1, D))

    Tg = _sc_gather(T, idx_flat)

    out = pl.pallas_call(
        _finish_kernel,
        grid=(NB,),
        in_specs=[
            pl.BlockSpec((BS * K, 2 * D), lambda i: (i, 0)),
            pl.BlockSpec((BS, 2 * D), lambda i: (i, 0)),
            pl.BlockSpec((BS, D), lambda i: (i, 0)),
            pl.BlockSpec((BS, D), lambda i: (i, 0)),
            pl.BlockSpec((BS, D), lambda i: (i, 0)),
            pl.BlockSpec((BS, D_EDGE), lambda i: (i, 0)),
            pl.BlockSpec((BS, D), lambda i: (i, 0)),
            pl.BlockSpec((D, D), lambda i: (0, 0)),
            pl.BlockSpec((D, D), lambda i: (0, 0)),
            pl.BlockSpec((D, D), lambda i: (0, 0)),
            pl.BlockSpec((D, D), lambda i: (0, 0)),
            pl.BlockSpec((D_EDGE, D), lambda i: (0, 0)),
            pl.BlockSpec((D, D), lambda i: (0, 0)),
            pl.BlockSpec((D, D), lambda i: (0, 0)),
            pl.BlockSpec((1, D), lambda i: (0, 0)),
            pl.BlockSpec((1, D), lambda i: (0, 0)),
        ],
        out_specs=pl.BlockSpec((BS, D), lambda i: (i, 0)),
        out_shape=jax.ShapeDtypeStruct((N, D), jnp.float32),
        compiler_params=pltpu.CompilerParams(
            dimension_semantics=("arbitrary",)),
    )(Tg, T, c, s, G, E, x, B1w, WaA2, WbA2, B2w, C2, W2d, W2e, const2, fc)

    return out


# speedup vs baseline: 3.0367x; 1.0577x over previous
import functools

import jax
import jax.numpy as jnp
from jax import lax
from jax.experimental import pallas as pl
from jax.experimental.pallas import tpu as pltpu
from jax.experimental.pallas import tpu_sc as plsc

N = 10000
K = 10
D = 128
D_EDGE = 20

BS = 400
NB = N // BS
NC, NS = 2, 16
NW = NC * NS

NPAD = 10240
JPW = NPAD // NW
JPC = 8
CH1 = JPC * K
NCH1 = JPW // JPC

B_PAD = NPAD * K
PER_W = B_PAD // NW
CH2 = 80
NCH2 = PER_W // CH2
NBUF = 4

SLOW_C = 0
CPP = 80
SLOW_CT = 24
FAST_CT = CPP - SLOW_CT


def _cos_poly(x):
    x2 = x * x
    return 1.0 + x2 * (-0.5 + x2 * (1.0 / 24 + x2 * (-1.0 / 720
                                                     + x2 * (1.0 / 40320))))


def _sin_poly(x):
    x2 = x * x
    return x * (1.0 + x2 * (-1.0 / 6 + x2 * (1.0 / 120 + x2 * (-1.0 / 5040
                                                               + x2 * (1.0 / 362880)))))


def _sc_gather_sum(table, idx2d):
    mesh = plsc.VectorSubcoreMesh(core_axis_name="c", subcore_axis_name="s")

    @functools.partial(
        pl.kernel,
        mesh=mesh,
        out_type=jax.ShapeDtypeStruct((NPAD, D), jnp.float32),
        scratch_types=[
            pltpu.VMEM((FAST_CT, CH1), jnp.int32),
            *[pltpu.VMEM((CH1, D), jnp.float32) for _ in range(NBUF)],
            *[pltpu.VMEM((JPC, D), jnp.float32) for _ in range(NBUF)],
            *[pltpu.SemaphoreType.DMA for _ in range(2 * NBUF)],
        ],
    )
    def gk(x_hbm, idx_hbm, g_hbm, idx_v, *bufs):
        rows = bufs[:NBUF]
        gbuf = bufs[NBUF:2 * NBUF]
        gsems = bufs[2 * NBUF:3 * NBUF]
        ssems = bufs[3 * NBUF:]
        c_ax = lax.axis_index("c")
        s_ax = lax.axis_index("s")
        is_slow = c_ax == SLOW_C
        count = jnp.where(is_slow, SLOW_CT, FAST_CT)
        cbase = s_ax * CPP + jnp.where(is_slow, FAST_CT, 0)

        @pl.when(is_slow)
        def _():
            pltpu.sync_copy(
                idx_hbm.at[pl.ds(pl.multiple_of(cbase, 8), SLOW_CT)],
                idx_v.at[pl.ds(0, SLOW_CT)])

        @pl.when(jnp.logical_not(is_slow))
        def _():
            pltpu.sync_copy(
                idx_hbm.at[pl.ds(pl.multiple_of(cbase, 8), FAST_CT)], idx_v)

        for b in range(NBUF):
            pltpu.async_copy(x_hbm.at[idx_v.at[b]], rows[b], gsems[b])

        def outer(g, carry):
            for b in range(NBUF):
                ci = g * NBUF + b
                pltpu.make_async_copy(
                    x_hbm.at[pl.ds(0, CH1)], rows[b], gsems[b]).wait()

                @pl.when(g > 0)
                def _():
                    pltpu.make_async_copy(
                        gbuf[b], g_hbm.at[pl.ds(0, JPC)], ssems[b]).wait()

                for jl in range(JPC):
                    for cc in range(D // 16):
                        sl = pl.ds(cc * 16, 16)
                        acc = rows[b][jl * K, sl]
                        for kk in range(1, K):
                            acc = acc + rows[b][jl * K + kk, sl]
                        gbuf[b][jl, sl] = acc
                pltpu.async_copy(
                    gbuf[b],
                    g_hbm.at[pl.ds(pl.multiple_of((cbase + ci) * JPC, 8),
                                   JPC)],
                    ssems[b])
                nci = ci + NBUF

                @pl.when(nci < count)
                def _():
                    pltpu.async_copy(
                        x_hbm.at[idx_v.at[nci]], rows[b], gsems[b])
            return carry

        lax.fori_loop(0, count // NBUF, outer, 0)
        for b in range(NBUF):
            pltpu.make_async_copy(
                gbuf[b], g_hbm.at[pl.ds(0, JPC)], ssems[b]).wait()

    return gk(table, idx2d)


def _sc_gather(table, idx1d):
    Wd = table.shape[1]
    mesh = plsc.VectorSubcoreMesh(core_axis_name="c", subcore_axis_name="s")

    @functools.partial(
        pl.kernel,
        mesh=mesh,
        out_type=jax.ShapeDtypeStruct((B_PAD, Wd), jnp.float32),
        scratch_types=[
            pltpu.VMEM((FAST_CT * CH2,), jnp.int32),
            *[pltpu.VMEM((CH2, Wd), jnp.float32) for _ in range(NBUF)],
            *[pltpu.SemaphoreType.DMA for _ in range(2 * NBUF)],
        ],
    )
    def gk(t_hbm, idx_hbm, out_hbm, idx_v, *bufs):
        rows = bufs[:NBUF]
        gsems = bufs[NBUF:2 * NBUF]
        ssems = bufs[2 * NBUF:]
        c_ax = lax.axis_index("c")
        s_ax = lax.axis_index("s")
        is_slow = c_ax == SLOW_C
        count = jnp.where(is_slow, SLOW_CT, FAST_CT)
        cbase = s_ax * CPP + jnp.where(is_slow, FAST_CT, 0)

        @pl.when(is_slow)
        def _():
            pltpu.sync_copy(
                idx_hbm.at[pl.ds(pl.multiple_of(cbase * CH2, 8),
                                 SLOW_CT * CH2)],
                idx_v.at[pl.ds(0, SLOW_CT * CH2)])

        @pl.when(jnp.logical_not(is_slow))
        def _():
            pltpu.sync_copy(
                idx_hbm.at[pl.ds(pl.multiple_of(cbase * CH2, 8),
                                 FAST_CT * CH2)], idx_v)

        for b in range(2):
            pltpu.async_copy(
                t_hbm.at[idx_v.at[pl.ds(b * CH2, CH2)]], rows[b], gsems[b])

        def outer(g, carry):
            for b in range(NBUF):
                ci = g * NBUF + b
                b2 = (b + 2) % NBUF
                nci = ci + 2
                pltpu.make_async_copy(
                    t_hbm.at[pl.ds(0, CH2)], rows[b], gsems[b]).wait()
                pltpu.async_copy(
                    rows[b],
                    out_hbm.at[pl.ds(
                        pl.multiple_of((cbase + ci) * CH2, 8), CH2)],
                    ssems[b])

                @pl.when(jnp.logical_and(nci >= NBUF, nci < count))
                def _():
                    pltpu.make_async_copy(
                        rows[b2], out_hbm.at[pl.ds(0, CH2)], ssems[b2]).wait()
                    pltpu.async_copy(
                        t_hbm.at[idx_v.at[pl.ds(pl.multiple_of(nci * CH2, 8),
                                                CH2)]],
                        rows[b2], gsems[b2])

                @pl.when(nci < NBUF)
                def _():
                    pltpu.async_copy(
                        t_hbm.at[idx_v.at[pl.ds(pl.multiple_of(nci * CH2, 8),
                                                CH2)]],
                        rows[b2], gsems[b2])
            return carry

        lax.fori_loop(0, count // NBUF, outer, 0)
        for b in range(NBUF):
            pltpu.make_async_copy(
                rows[b], out_hbm.at[pl.ds(0, CH2)], ssems[b]).wait()

    return gk(table, idx1d)


def _tables_kernel(G_ref, et_ref, ef_ref, t_ref, w_ref, b_ref, A1_ref, C1_ref,
                   b1_ref, T_ref, c_ref, s_ref, E_ref):
    w = w_ref[...]
    Cacc = jnp.zeros((BS, D), jnp.float32)
    Sacc = jnp.zeros((BS, D), jnp.float32)
    for kk in range(K):
        ang = et_ref[:, kk:kk + 1] * w
        Cacc = Cacc + _cos_poly(ang)
        Sacc = Sacc + _sin_poly(ang)
    Eacc = jnp.zeros((BS, D_EDGE), jnp.float32)
    for kk in range(K):
        Eacc = Eacc + ef_ref[:, kk * D_EDGE:(kk + 1) * D_EDGE]
    P = (jnp.dot(G_ref[...], A1_ref[...], preferred_element_type=jnp.float32)
         + jnp.dot(Eacc, C1_ref[...], preferred_element_type=jnp.float32)
         + float(K) * b1_ref[...])
    cu = ((jax.lax.bitcast_convert_type(Cacc, jnp.uint32) + jnp.uint32(0x8000))
          & jnp.uint32(0xFFFF0000))
    su = ((jax.lax.bitcast_convert_type(Sacc, jnp.uint32) + jnp.uint32(0x8000))
          >> 16)
    T_ref[:, :D] = jax.lax.bitcast_convert_type(cu | su, jnp.float32)
    T_ref[:, D:] = P
    phase = t_ref[...] * w + b_ref[...]
    c_ref[...] = _cos_poly(phase)
    s_ref[...] = _sin_poly(phase)
    E_ref[...] = Eacc


def _finish_kernel(Tg_ref, T_ref, c_ref, s_ref, G_ref, E_ref, x_ref,
                   B1_ref, WaA2_ref, WbA2_ref, B2_ref, C2_ref, W2d_ref,
                   W2e_ref, const2_ref, fc_ref, out_ref):
    c = c_ref[...]
    s = s_ref[...]
    c4 = jnp.broadcast_to(c[:, None, :], (BS, K, D)).reshape(BS * K, D)
    s4 = jnp.broadcast_to(s[:, None, :], (BS, K, D)).reshape(BS * K, D)
    wg = jax.lax.bitcast_convert_type(Tg_ref[:, :D], jnp.uint32)
    Cg = jax.lax.bitcast_convert_type(wg & jnp.uint32(0xFFFF0000), jnp.float32)
    Sg = jax.lax.bitcast_convert_type(wg << 16, jnp.float32)
    Pg = Tg_ref[:, D:]
    vm = c4 * Cg + s4 * Sg
    u = jnp.maximum(
        Pg + jnp.dot(vm, B1_ref[...], preferred_element_type=jnp.float32), 0.0)
    U = jnp.sum(u.reshape(BS, K, D), axis=1)
    wt = jax.lax.bitcast_convert_type(T_ref[:, :D], jnp.uint32)
    tt = (c * jax.lax.bitcast_convert_type(wt & jnp.uint32(0xFFFF0000),
                                           jnp.float32)
          + s * jax.lax.bitcast_convert_type(wt << 16, jnp.float32))
    pre = (jnp.dot(U, WaA2_ref[...], preferred_element_type=jnp.float32)
           + jnp.dot(G_ref[...], WbA2_ref[...], preferred_element_type=jnp.float32)
           + jnp.dot(tt, B2_ref[...], preferred_element_type=jnp.float32)
           + jnp.dot(E_ref[...], C2_ref[...], preferred_element_type=jnp.float32)
           + const2_ref[...])
    out_ref[...] = (
        jnp.dot(jnp.maximum(pre, 0.0), W2d_ref[...],
                preferred_element_type=jnp.float32)
        + jnp.dot(x_ref[...], W2e_ref[...], preferred_element_type=jnp.float32)
        + fc_ref[...])


def kernel(x, t, neighbor_idx, edge_times, edge_feats, time_w, time_b, W1, b1, W2, b2):
    nbr = neighbor_idx.astype(jnp.int32)
    pad = jnp.zeros((NPAD * K - N * K,), jnp.int32)
    idx_flat = jnp.concatenate([nbr.reshape(-1), pad])
    idx_j = idx_flat.reshape(NW * NCH1, CH1)

    A1, B1w, C1 = W1[0][:D], W1[0][D:2 * D], W1[0][2 * D:]
    A2, B2w, C2 = W1[1][:D], W1[1][D:2 * D], W1[1][2 * D:]
    W2a, W2b, W2c = W2[0][:D], W2[0][D:2 * D], W2[0][2 * D:]
    W2d, W2e, W2f = W2[1][:D], W2[1][D:2 * D], W2[1][2 * D:]
    z = jnp.cos(time_b)
    cr = z @ W2c + b2[0]
    WaA2 = W2a @ A2
    WbA2 = W2b @ A2
    const2 = (float(K) * (cr @ A2 + b1[1])).reshape(1, D)
    fc = (z @ W2f + b2[1]).reshape(1, D)

    G = _sc_gather_sum(x, idx_j)

    ef2 = edge_feats.reshape(N, K * D_EDGE)
    T, c, s, E = pl.pallas_call(
        _tables_kernel,
        grid=(NB,),
        in_specs=[
            pl.BlockSpec((BS, D), lambda i: (i, 0)),
            pl.BlockSpec((BS, K), lambda i: (i, 0)),
            pl.BlockSpec((BS, K * D_EDGE), lambda i: (i, 0)),
            pl.BlockSpec((BS, 1), lambda i: (i, 0)),
            pl.BlockSpec((1, D), lambda i: (0, 0)),
            pl.BlockSpec((1, D), lambda i: (0, 0)),
            pl.BlockSpec((D, D), lambda i: (0, 0)),
            pl.BlockSpec((D_EDGE, D), lambda i: (0, 0)),
            pl.BlockSpec((1, D), lambda i: (0, 0)),
        ],
        out_specs=[
            pl.BlockSpec((BS, 2 * D), lambda i: (i, 0)),
            pl.BlockSpec((BS, D), lambda i: (i, 0)),
            pl.BlockSpec((BS, D), lambda i: (i, 0)),
            pl.BlockSpec((BS, D_EDGE), lambda i: (i, 0)),
        ],
        out_shape=[
            jax.ShapeDtypeStruct((N, 2 * D), jnp.float32),
            jax.ShapeDtypeStruct((N, D), jnp.float32),
            jax.ShapeDtypeStruct((N, D), jnp.float32),
            jax.ShapeDtypeStruct((N, D_EDGE), jnp.float32),
        ],
        compiler_params=pltpu.CompilerParams(
            dimension_semantics=("arbitrary",)),
    )(G, edge_times, ef2, t.reshape(N, 1), time_w.reshape(1, D),
      time_b.reshape(1, D), A1, C1, b1[0].reshape(1, D))

    Tg = _sc_gather(T, idx_flat)

    out = pl.pallas_call(
        _finish_kernel,
        grid=(NB,),
        in_specs=[
            pl.BlockSpec((BS * K, 2 * D), lambda i: (i, 0)),
            pl.BlockSpec((BS, 2 * D), lambda i: (i, 0)),
            pl.BlockSpec((BS, D), lambda i: (i, 0)),
            pl.BlockSpec((BS, D), lambda i: (i, 0)),
            pl.BlockSpec((BS, D), lambda i: (i, 0)),
            pl.BlockSpec((BS, D_EDGE), lambda i: (i, 0)),
            pl.BlockSpec((BS, D), lambda i: (i, 0)),
            pl.BlockSpec((D, D), lambda i: (0, 0)),
            pl.BlockSpec((D, D), lambda i: (0, 0)),
            pl.BlockSpec((D, D), lambda i: (0, 0)),
            pl.BlockSpec((D, D), lambda i: (0, 0)),
            pl.BlockSpec((D_EDGE, D), lambda i: (0, 0)),
            pl.BlockSpec((D, D), lambda i: (0, 0)),
            pl.BlockSpec((D, D), lambda i: (0, 0)),
            pl.BlockSpec((1, D), lambda i: (0, 0)),
            pl.BlockSpec((1, D), lambda i: (0, 0)),
        ],
        out_specs=pl.BlockSpec((BS, D), lambda i: (i, 0)),
        out_shape=jax.ShapeDtypeStruct((N, D), jnp.float32),
        compiler_params=pltpu.CompilerParams(
            dimension_semantics=("arbitrary",)),
    )(Tg, T, c, s, G, E, x, B1w, WaA2, WbA2, B2w, C2, W2d, W2e, const2, fc)

    return out


# speedup vs baseline: 3.1905x; 1.0506x over previous
import functools

import jax
import jax.numpy as jnp
from jax import lax
from jax.experimental import pallas as pl
from jax.experimental.pallas import tpu as pltpu
from jax.experimental.pallas import tpu_sc as plsc

N = 10000
K = 10
D = 128
D_EDGE = 20

BS = 400
NB = N // BS
NC, NS = 2, 16
NW = NC * NS

NPAD = 10240
JPW = NPAD // NW
JPC = 8
CH1 = JPC * K
NCH1 = JPW // JPC

B_PAD = NPAD * K
PER_W = B_PAD // NW
CH2 = 80
NCH2 = PER_W // CH2
NBUF = 4

SLOW_C = 0
CPP = 80
SLOW_CT = 16
FAST_CT = CPP - SLOW_CT


def _cos_poly(x):
    x2 = x * x
    return 1.0 + x2 * (-0.5 + x2 * (1.0 / 24 + x2 * (-1.0 / 720
                                                     + x2 * (1.0 / 40320))))


def _sin_poly(x):
    x2 = x * x
    return x * (1.0 + x2 * (-1.0 / 6 + x2 * (1.0 / 120 + x2 * (-1.0 / 5040
                                                               + x2 * (1.0 / 362880)))))


def _sc_gather_sum(table, idx2d):
    mesh = plsc.VectorSubcoreMesh(core_axis_name="c", subcore_axis_name="s")

    @functools.partial(
        pl.kernel,
        mesh=mesh,
        out_type=jax.ShapeDtypeStruct((NPAD, D), jnp.float32),
        scratch_types=[
            pltpu.VMEM((FAST_CT, CH1), jnp.int32),
            *[pltpu.VMEM((CH1, D), jnp.float32) for _ in range(NBUF)],
            *[pltpu.VMEM((JPC, D), jnp.float32) for _ in range(NBUF)],
            *[pltpu.SemaphoreType.DMA for _ in range(2 * NBUF)],
        ],
    )
    def gk(x_hbm, idx_hbm, g_hbm, idx_v, *bufs):
        rows = bufs[:NBUF]
        gbuf = bufs[NBUF:2 * NBUF]
        gsems = bufs[2 * NBUF:3 * NBUF]
        ssems = bufs[3 * NBUF:]
        c_ax = lax.axis_index("c")
        s_ax = lax.axis_index("s")
        is_slow = c_ax == SLOW_C
        count = jnp.where(is_slow, SLOW_CT, FAST_CT)
        cbase = s_ax * CPP + jnp.where(is_slow, FAST_CT, 0)

        @pl.when(is_slow)
        def _():
            pltpu.sync_copy(
                idx_hbm.at[pl.ds(pl.multiple_of(cbase, 8), SLOW_CT)],
                idx_v.at[pl.ds(0, SLOW_CT)])

        @pl.when(jnp.logical_not(is_slow))
        def _():
            pltpu.sync_copy(
                idx_hbm.at[pl.ds(pl.multiple_of(cbase, 8), FAST_CT)], idx_v)

        for b in range(NBUF):
            pltpu.async_copy(x_hbm.at[idx_v.at[b]], rows[b], gsems[b])

        def outer(g, carry):
            for b in range(NBUF):
                ci = g * NBUF + b
                pltpu.make_async_copy(
                    x_hbm.at[pl.ds(0, CH1)], rows[b], gsems[b]).wait()

                @pl.when(g > 0)
                def _():
                    pltpu.make_async_copy(
                        gbuf[b], g_hbm.at[pl.ds(0, JPC)], ssems[b]).wait()

                for jl in range(JPC):
                    for cc in range(D // 16):
                        sl = pl.ds(cc * 16, 16)
                        acc = rows[b][jl * K, sl]
                        for kk in range(1, K):
                            acc = acc + rows[b][jl * K + kk, sl]
                        gbuf[b][jl, sl] = acc
                pltpu.async_copy(
                    gbuf[b],
                    g_hbm.at[pl.ds(pl.multiple_of((cbase + ci) * JPC, 8),
                                   JPC)],
                    ssems[b])
                nci = ci + NBUF

                @pl.when(nci < count)
                def _():
                    pltpu.async_copy(
                        x_hbm.at[idx_v.at[nci]], rows[b], gsems[b])
            return carry

        lax.fori_loop(0, count // NBUF, outer, 0)
        for b in range(NBUF):
            pltpu.make_async_copy(
                gbuf[b], g_hbm.at[pl.ds(0, JPC)], ssems[b]).wait()

    return gk(table, idx2d)


def _sc_gather(table, idx1d):
    Wd = table.shape[1]
    mesh = plsc.VectorSubcoreMesh(core_axis_name="c", subcore_axis_name="s")

    @functools.partial(
        pl.kernel,
        mesh=mesh,
        out_type=jax.ShapeDtypeStruct((B_PAD, Wd), jnp.float32),
        scratch_types=[
            pltpu.VMEM((FAST_CT * CH2,), jnp.int32),
            *[pltpu.VMEM((CH2, Wd), jnp.float32) for _ in range(NBUF)],
            *[pltpu.SemaphoreType.DMA for _ in range(2 * NBUF)],
        ],
    )
    def gk(t_hbm, idx_hbm, out_hbm, idx_v, *bufs):
        rows = bufs[:NBUF]
        gsems = bufs[NBUF:2 * NBUF]
        ssems = bufs[2 * NBUF:]
        c_ax = lax.axis_index("c")
        s_ax = lax.axis_index("s")
        is_slow = c_ax == SLOW_C
        count = jnp.where(is_slow, SLOW_CT, FAST_CT)
        cbase = s_ax * CPP + jnp.where(is_slow, FAST_CT, 0)

        @pl.when(is_slow)
        def _():
            pltpu.sync_copy(
                idx_hbm.at[pl.ds(pl.multiple_of(cbase * CH2, 8),
                                 SLOW_CT * CH2)],
                idx_v.at[pl.ds(0, SLOW_CT * CH2)])

        @pl.when(jnp.logical_not(is_slow))
        def _():
            pltpu.sync_copy(
                idx_hbm.at[pl.ds(pl.multiple_of(cbase * CH2, 8),
                                 FAST_CT * CH2)], idx_v)

        for b in range(2):
            pltpu.async_copy(
                t_hbm.at[idx_v.at[pl.ds(b * CH2, CH2)]], rows[b], gsems[b])

        def outer(g, carry):
            for b in range(NBUF):
                ci = g * NBUF + b
                b2 = (b + 2) % NBUF
                nci = ci + 2
                pltpu.make_async_copy(
                    t_hbm.at[pl.ds(0, CH2)], rows[b], gsems[b]).wait()
                pltpu.async_copy(
                    rows[b],
                    out_hbm.at[pl.ds(
                        pl.multiple_of((cbase + ci) * CH2, 8), CH2)],
                    ssems[b])

                @pl.when(jnp.logical_and(nci >= NBUF, nci < count))
                def _():
                    pltpu.make_async_copy(
                        rows[b2], out_hbm.at[pl.ds(0, CH2)], ssems[b2]).wait()
                    pltpu.async_copy(
                        t_hbm.at[idx_v.at[pl.ds(pl.multiple_of(nci * CH2, 8),
                                                CH2)]],
                        rows[b2], gsems[b2])

                @pl.when(nci < NBUF)
                def _():
                    pltpu.async_copy(
                        t_hbm.at[idx_v.at[pl.ds(pl.multiple_of(nci * CH2, 8),
                                                CH2)]],
                        rows[b2], gsems[b2])
            return carry

        lax.fori_loop(0, count // NBUF, outer, 0)
        for b in range(NBUF):
            pltpu.make_async_copy(
                rows[b], out_hbm.at[pl.ds(0, CH2)], ssems[b]).wait()

    return gk(table, idx1d)


def _cs_kernel(et_ref, ef_ref, t_ref, w_ref, b_ref, CS_ref, c_ref, s_ref,
               E_ref):
    w = w_ref[...]
    Cacc = jnp.zeros((BS, D), jnp.float32)
    Sacc = jnp.zeros((BS, D), jnp.float32)
    for kk in range(K):
        ang = et_ref[:, kk:kk + 1] * w
        Cacc = Cacc + _cos_poly(ang)
        Sacc = Sacc + _sin_poly(ang)
    Eacc = jnp.zeros((BS, D_EDGE), jnp.float32)
    for kk in range(K):
        Eacc = Eacc + ef_ref[:, kk * D_EDGE:(kk + 1) * D_EDGE]
    cu = ((jax.lax.bitcast_convert_type(Cacc, jnp.uint32) + jnp.uint32(0x8000))
          & jnp.uint32(0xFFFF0000))
    su = ((jax.lax.bitcast_convert_type(Sacc, jnp.uint32) + jnp.uint32(0x8000))
          >> 16)
    CS_ref[...] = jax.lax.bitcast_convert_type(cu | su, jnp.float32)
    phase = t_ref[...] * w + b_ref[...]
    c_ref[...] = _cos_poly(phase)
    s_ref[...] = _sin_poly(phase)
    E_ref[...] = Eacc


def _p_kernel(G_ref, E_ref, CS_ref, A1_ref, C1_ref, b1_ref, T_ref):
    T_ref[:, :D] = CS_ref[...]
    T_ref[:, D:] = (
        jnp.dot(G_ref[...], A1_ref[...], preferred_element_type=jnp.float32)
        + jnp.dot(E_ref[...], C1_ref[...], preferred_element_type=jnp.float32)
        + float(K) * b1_ref[...])


def _finish_kernel(Tg_ref, T_ref, c_ref, s_ref, G_ref, E_ref, x_ref,
                   B1_ref, WaA2_ref, WbA2_ref, B2_ref, C2_ref, W2d_ref,
                   W2e_ref, const2_ref, fc_ref, out_ref):
    c = c_ref[...]
    s = s_ref[...]
    c4 = jnp.broadcast_to(c[:, None, :], (BS, K, D)).reshape(BS * K, D)
    s4 = jnp.broadcast_to(s[:, None, :], (BS, K, D)).reshape(BS * K, D)
    wg = jax.lax.bitcast_convert_type(Tg_ref[:, :D], jnp.uint32)
    Cg = jax.lax.bitcast_convert_type(wg & jnp.uint32(0xFFFF0000), jnp.float32)
    Sg = jax.lax.bitcast_convert_type(wg << 16, jnp.float32)
    Pg = Tg_ref[:, D:]
    vm = c4 * Cg + s4 * Sg
    u = jnp.maximum(
        Pg + jnp.dot(vm, B1_ref[...], preferred_element_type=jnp.float32), 0.0)
    U = jnp.sum(u.reshape(BS, K, D), axis=1)
    wt = jax.lax.bitcast_convert_type(T_ref[:, :D], jnp.uint32)
    tt = (c * jax.lax.bitcast_convert_type(wt & jnp.uint32(0xFFFF0000),
                                           jnp.float32)
          + s * jax.lax.bitcast_convert_type(wt << 16, jnp.float32))
    pre = (jnp.dot(U, WaA2_ref[...], preferred_element_type=jnp.float32)
           + jnp.dot(G_ref[...], WbA2_ref[...], preferred_element_type=jnp.float32)
           + jnp.dot(tt, B2_ref[...], preferred_element_type=jnp.float32)
           + jnp.dot(E_ref[...], C2_ref[...], preferred_element_type=jnp.float32)
           + const2_ref[...])
    out_ref[...] = (
        jnp.dot(jnp.maximum(pre, 0.0), W2d_ref[...],
                preferred_element_type=jnp.float32)
        + jnp.dot(x_ref[...], W2e_ref[...], preferred_element_type=jnp.float32)
        + fc_ref[...])


def kernel(x, t, neighbor_idx, edge_times, edge_feats, time_w, time_b, W1, b1, W2, b2):
    nbr = neighbor_idx.astype(jnp.int32)
    pad = jnp.zeros((NPAD * K - N * K,), jnp.int32)
    idx_flat = jnp.concatenate([nbr.reshape(-1), pad])
    idx_j = idx_flat.reshape(NW * NCH1, CH1)

    A1, B1w, C1 = W1[0][:D], W1[0][D:2 * D], W1[0][2 * D:]
    A2, B2w, C2 = W1[1][:D], W1[1][D:2 * D], W1[1][2 * D:]
    W2a, W2b, W2c = W2[0][:D], W2[0][D:2 * D], W2[0][2 * D:]
    W2d, W2e, W2f = W2[1][:D], W2[1][D:2 * D], W2[1][2 * D:]
    z = jnp.cos(time_b)
    cr = z @ W2c + b2[0]
    WaA2 = W2a @ A2
    WbA2 = W2b @ A2
    const2 = (float(K) * (cr @ A2 + b1[1])).reshape(1, D)
    fc = (z @ W2f + b2[1]).reshape(1, D)

    G = _sc_gather_sum(x, idx_j)

    ef2 = edge_feats.reshape(N, K * D_EDGE)
    CS, c, s, E = pl.pallas_call(
        _cs_kernel,
        grid=(NB,),
        in_specs=[
            pl.BlockSpec((BS, K), lambda i: (i, 0)),
            pl.BlockSpec((BS, K * D_EDGE), lambda i: (i, 0)),
            pl.BlockSpec((BS, 1), lambda i: (i, 0)),
            pl.BlockSpec((1, D), lambda i: (0, 0)),
            pl.BlockSpec((1, D), lambda i: (0, 0)),
        ],
        out_specs=[
            pl.BlockSpec((BS, D), lambda i: (i, 0)),
            pl.BlockSpec((BS, D), lambda i: (i, 0)),
            pl.BlockSpec((BS, D), lambda i: (i, 0)),
            pl.BlockSpec((BS, D_EDGE), lambda i: (i, 0)),
        ],
        out_shape=[
            jax.ShapeDtypeStruct((N, D), jnp.float32),
            jax.ShapeDtypeStruct((N, D), jnp.float32),
            jax.ShapeDtypeStruct((N, D), jnp.float32),
            jax.ShapeDtypeStruct((N, D_EDGE), jnp.float32),
        ],
        compiler_params=pltpu.CompilerParams(
            dimension_semantics=("arbitrary",)),
    )(edge_times, ef2, t.reshape(N, 1), time_w.reshape(1, D),
      time_b.reshape(1, D))

    T = pl.pallas_call(
        _p_kernel,
        grid=(NB,),
        in_specs=[
            pl.BlockSpec((BS, D), lambda i: (i, 0)),
            pl.BlockSpec((BS, D_EDGE), lambda i: (i, 0)),
            pl.BlockSpec((BS, D), lambda i: (i, 0)),
            pl.BlockSpec((D, D), lambda i: (0, 0)),
            pl.BlockSpec((D_EDGE, D), lambda i: (0, 0)),
            pl.BlockSpec((1, D), lambda i: (0, 0)),
        ],
        out_specs=pl.BlockSpec((BS, 2 * D), lambda i: (i, 0)),
        out_shape=jax.ShapeDtypeStruct((N, 2 * D), jnp.float32),
        compiler_params=pltpu.CompilerParams(
            dimension_semantics=("arbitrary",)),
    )(G, E, CS, A1, C1, b1[0].reshape(1, D))

    Tg = _sc_gather(T, idx_flat)

    out = pl.pallas_call(
        _finish_kernel,
        grid=(NB,),
        in_specs=[
            pl.BlockSpec((BS * K, 2 * D), lambda i: (i, 0)),
            pl.BlockSpec((BS, 2 * D), lambda i: (i, 0)),
            pl.BlockSpec((BS, D), lambda i: (i, 0)),
            pl.BlockSpec((BS, D), lambda i: (i, 0)),
            pl.BlockSpec((BS, D), lambda i: (i, 0)),
            pl.BlockSpec((BS, D_EDGE), lambda i: (i, 0)),
            pl.BlockSpec((BS, D), lambda i: (i, 0)),
            pl.BlockSpec((D, D), lambda i: (0, 0)),
            pl.BlockSpec((D, D), lambda i: (0, 0)),
            pl.BlockSpec((D, D), lambda i: (0, 0)),
            pl.BlockSpec((D, D), lambda i: (0, 0)),
            pl.BlockSpec((D_EDGE, D), lambda i: (0, 0)),
            pl.BlockSpec((D, D), lambda i: (0, 0)),
            pl.BlockSpec((D, D), lambda i: (0, 0)),
            pl.BlockSpec((1, D), lambda i: (0, 0)),
            pl.BlockSpec((1, D), lambda i: (0, 0)),
        ],
        out_specs=pl.BlockSpec((BS, D), lambda i: (i, 0)),
        out_shape=jax.ShapeDtypeStruct((N, D), jnp.float32),
        compiler_params=pltpu.CompilerParams(
            dimension_semantics=("arbitrary",)),
    )(Tg, T, c, s, G, E, x, B1w, WaA2, WbA2, B2w, C2, W2d, W2e, const2, fc)

    return out
